# baseline (speedup 1.0000x reference)
import functools

import jax
import jax.numpy as jnp
from jax import lax
from jax.experimental import pallas as pl
from jax.experimental.pallas import tpu as pltpu
from jax.experimental.pallas import tpu_sc as plsc

_S, _B, _D = 8192, 2, 1024
_NC, _NS = 2, 16
_NW = _NC * _NS
_KSC = 3584
_PW = _KSC // _NW
_C = 8
_NCH = _PW // _C
_LANES = 16
_TC_SBLK = 512


def _sc_body(emb_hbm, pe_hbm, out_hbm,
             emb_v0, emb_v1, pe_v0, pe_v1, out_v0, out_v1,
             sin0, sin1, sout0, sout1):
    wid = lax.axis_index("s") * _NC + lax.axis_index("c")
    base = wid * _PW
    emb_bufs = (emb_v0, emb_v1)
    pe_bufs = (pe_v0, pe_v1)
    out_bufs = (out_v0, out_v1)
    sins = (sin0, sin1)
    souts = (sout0, sout1)

    def start_in(g, b):
        s0 = base + g * _C
        pltpu.async_copy(emb_hbm.at[pl.ds(s0, _C)], emb_bufs[b], sins[b])
        pltpu.async_copy(pe_hbm.at[pl.ds(s0, _C)], pe_bufs[b], sins[b])

    def wait_in(b):
        pltpu.make_async_copy(emb_hbm.at[pl.ds(base, _C)], emb_bufs[b], sins[b]).wait()
        pltpu.make_async_copy(pe_hbm.at[pl.ds(base, _C)], pe_bufs[b], sins[b]).wait()

    def start_out(g, b):
        s0 = base + g * _C
        pltpu.async_copy(out_bufs[b], out_hbm.at[pl.ds(s0, _C)], souts[b])

    def wait_out(b):
        pltpu.make_async_copy(out_bufs[b], out_hbm.at[pl.ds(base, _C)], souts[b]).wait()

    start_in(0, 0)
    start_in(1, 1)

    def outer(k, _):
        for b in range(2):
            g = 2 * k + b
            wait_in(b)

            @pl.when(g >= 2)
            def _():
                wait_out(b)

            @plsc.parallel_loop(0, _D // _LANES, unroll=4)
            def lane(j):
                off = j * _LANES
                for i in range(_C):
                    pe_vec = pe_bufs[b][i, pl.ds(off, _LANES)]
                    out_bufs[b][i, 0, pl.ds(off, _LANES)] = (
                        emb_bufs[b][i, 0, pl.ds(off, _LANES)] + pe_vec)
                    out_bufs[b][i, 1, pl.ds(off, _LANES)] = (
                        emb_bufs[b][i, 1, pl.ds(off, _LANES)] + pe_vec)

            start_out(g, b)

            @pl.when(g + 2 < _NCH)
            def _():
                start_in(g + 2, b)
        return 0

    lax.fori_loop(0, _NCH // 2, outer, 0)
    wait_out(0)
    wait_out(1)


def _tc_add_kernel(emb_ref, pe_ref, out_ref):
    pe = pe_ref[...]
    out_ref[...] = emb_ref[...] + pe[:, None, :]


def kernel(emb, pe_table):
    sc_kernel = pl.kernel(
        _sc_body,
        out_type=jax.ShapeDtypeStruct((_KSC, _B, _D), jnp.float32),
        mesh=plsc.VectorSubcoreMesh(core_axis_name="c", subcore_axis_name="s"),
        scratch_types=[
            pltpu.VMEM((_C, _B, _D), jnp.float32),
            pltpu.VMEM((_C, _B, _D), jnp.float32),
            pltpu.VMEM((_C, _D), jnp.float32),
            pltpu.VMEM((_C, _D), jnp.float32),
            pltpu.VMEM((_C, _B, _D), jnp.float32),
            pltpu.VMEM((_C, _B, _D), jnp.float32),
            pltpu.SemaphoreType.DMA,
            pltpu.SemaphoreType.DMA,
            pltpu.SemaphoreType.DMA,
            pltpu.SemaphoreType.DMA,
        ],
    )
    sc_out = sc_kernel(emb, pe_table)

    n_tc_blocks = (_S - _KSC) // _TC_SBLK
    off_blocks = _KSC // _TC_SBLK
    tc_out = pl.pallas_call(
        _tc_add_kernel,
        grid=(n_tc_blocks,),
        in_specs=[
            pl.BlockSpec((_TC_SBLK, _B, _D), lambda i: (i + off_blocks, 0, 0)),
            pl.BlockSpec((_TC_SBLK, _D), lambda i: (i + off_blocks, 0)),
        ],
        out_specs=pl.BlockSpec((_TC_SBLK, _B, _D), lambda i: (i + off_blocks, 0, 0)),
        out_shape=jax.ShapeDtypeStruct((_S, _B, _D), jnp.float32),
    )(emb, pe_table)

    return lax.dynamic_update_slice(tc_out, sc_out, (0, 0, 0))


# speedup vs baseline: 1.1137x; 1.1137x over previous
import functools

import jax
import jax.numpy as jnp
from jax import lax
from jax.experimental import pallas as pl
from jax.experimental.pallas import tpu as pltpu
from jax.experimental.pallas import tpu_sc as plsc

_S, _B, _D = 8192, 2, 1024
_NC, _NS = 2, 16
_NW = _NC * _NS
_KSC = 1024
_PW = _KSC // _NW
_C = 8
_NCH = _PW // _C
_LANES = 16
_TC_SBLK = 512


def _sc_body(emb_hbm, pe_hbm, out_hbm,
             emb_v0, emb_v1, pe_v0, pe_v1, out_v0, out_v1,
             sin0, sin1, sout0, sout1):
    wid = lax.axis_index("s") * _NC + lax.axis_index("c")
    base = wid * _PW
    emb_bufs = (emb_v0, emb_v1)
    pe_bufs = (pe_v0, pe_v1)
    out_bufs = (out_v0, out_v1)
    sins = (sin0, sin1)
    souts = (sout0, sout1)

    def start_in(g, b):
        s0 = base + g * _C
        pltpu.async_copy(emb_hbm.at[pl.ds(s0, _C)], emb_bufs[b], sins[b])
        pltpu.async_copy(pe_hbm.at[pl.ds(s0, _C)], pe_bufs[b], sins[b])

    def wait_in(b):
        pltpu.make_async_copy(emb_hbm.at[pl.ds(base, _C)], emb_bufs[b], sins[b]).wait()
        pltpu.make_async_copy(pe_hbm.at[pl.ds(base, _C)], pe_bufs[b], sins[b]).wait()

    def start_out(g, b):
        s0 = base + g * _C
        pltpu.async_copy(out_bufs[b], out_hbm.at[pl.ds(s0, _C)], souts[b])

    def wait_out(b):
        pltpu.make_async_copy(out_bufs[b], out_hbm.at[pl.ds(base, _C)], souts[b]).wait()

    start_in(0, 0)
    start_in(1, 1)

    def outer(k, _):
        for b in range(2):
            g = 2 * k + b
            wait_in(b)

            @pl.when(g >= 2)
            def _():
                wait_out(b)

            @plsc.parallel_loop(0, _D // _LANES, unroll=4)
            def lane(j):
                off = j * _LANES
                for i in range(_C):
                    pe_vec = pe_bufs[b][i, pl.ds(off, _LANES)]
                    out_bufs[b][i, 0, pl.ds(off, _LANES)] = (
                        emb_bufs[b][i, 0, pl.ds(off, _LANES)] + pe_vec)
                    out_bufs[b][i, 1, pl.ds(off, _LANES)] = (
                        emb_bufs[b][i, 1, pl.ds(off, _LANES)] + pe_vec)

            start_out(g, b)

            @pl.when(g + 2 < _NCH)
            def _():
                start_in(g + 2, b)
        return 0

    lax.fori_loop(0, _NCH // 2, outer, 0)
    wait_out(0)
    wait_out(1)


def _tc_add_kernel(emb_ref, pe_ref, out_ref):
    pe = pe_ref[...]
    out_ref[...] = emb_ref[...] + pe[:, None, :]


def kernel(emb, pe_table):
    sc_kernel = pl.kernel(
        _sc_body,
        out_type=jax.ShapeDtypeStruct((_KSC, _B, _D), jnp.float32),
        mesh=plsc.VectorSubcoreMesh(core_axis_name="c", subcore_axis_name="s"),
        scratch_types=[
            pltpu.VMEM((_C, _B, _D), jnp.float32),
            pltpu.VMEM((_C, _B, _D), jnp.float32),
            pltpu.VMEM((_C, _D), jnp.float32),
            pltpu.VMEM((_C, _D), jnp.float32),
            pltpu.VMEM((_C, _B, _D), jnp.float32),
            pltpu.VMEM((_C, _B, _D), jnp.float32),
            pltpu.SemaphoreType.DMA,
            pltpu.SemaphoreType.DMA,
            pltpu.SemaphoreType.DMA,
            pltpu.SemaphoreType.DMA,
        ],
    )
    sc_out = sc_kernel(emb, pe_table)

    n_tc_blocks = (_S - _KSC) // _TC_SBLK
    off_blocks = _KSC // _TC_SBLK
    tc_out = pl.pallas_call(
        _tc_add_kernel,
        grid=(n_tc_blocks,),
        in_specs=[
            pl.BlockSpec((_TC_SBLK, _B, _D), lambda i: (i + off_blocks, 0, 0)),
            pl.BlockSpec((_TC_SBLK, _D), lambda i: (i + off_blocks, 0)),
        ],
        out_specs=pl.BlockSpec((_TC_SBLK, _B, _D), lambda i: (i + off_blocks, 0, 0)),
        out_shape=jax.ShapeDtypeStruct((_S, _B, _D), jnp.float32),
    )(emb, pe_table)

    return lax.dynamic_update_slice(tc_out, sc_out, (0, 0, 0))


# speedup vs baseline: 1.1746x; 1.0547x over previous
import functools

import jax
import jax.numpy as jnp
from jax import lax
from jax.experimental import pallas as pl
from jax.experimental.pallas import tpu as pltpu
from jax.experimental.pallas import tpu_sc as plsc

_S, _B, _D = 8192, 2, 1024
_NC, _NS = 2, 16
_NW = _NC * _NS
_PW = _S // _NW
_C = 4
_NCH = _PW // _C
_NBUF = 4
_LANES = 16


def _sc_body(emb_hbm, pe_hbm, out_hbm, emb_bufs, pe_bufs, out_bufs, sins, souts):
    wid = lax.axis_index("s") * _NC + lax.axis_index("c")
    base = wid * _PW

    def start_in(g, b):
        s0 = base + g * _C
        pltpu.async_copy(emb_hbm.at[pl.ds(s0, _C)], emb_bufs[b], sins[b])
        pltpu.async_copy(pe_hbm.at[pl.ds(s0, _C)], pe_bufs[b], sins[b])

    def wait_in(b):
        pltpu.make_async_copy(emb_hbm.at[pl.ds(base, _C)], emb_bufs[b], sins[b]).wait()
        pltpu.make_async_copy(pe_hbm.at[pl.ds(base, _C)], pe_bufs[b], sins[b]).wait()

    def start_out(g, b):
        s0 = base + g * _C
        pltpu.async_copy(out_bufs[b], out_hbm.at[pl.ds(s0, _C)], souts[b])

    def wait_out(b):
        pltpu.make_async_copy(out_bufs[b], out_hbm.at[pl.ds(base, _C)], souts[b]).wait()

    for b in range(_NBUF):
        start_in(b, b)

    def outer(k, _):
        for b in range(_NBUF):
            g = _NBUF * k + b
            wait_in(b)

            @pl.when(g >= _NBUF)
            def _():
                wait_out(b)

            @plsc.parallel_loop(0, _D // _LANES, unroll=8)
            def lane(j):
                off = j * _LANES
                for i in range(_C):
                    pe_vec = pe_bufs[b][i, pl.ds(off, _LANES)]
                    out_bufs[b][i, 0, pl.ds(off, _LANES)] = (
                        emb_bufs[b][i, 0, pl.ds(off, _LANES)] + pe_vec)
                    out_bufs[b][i, 1, pl.ds(off, _LANES)] = (
                        emb_bufs[b][i, 1, pl.ds(off, _LANES)] + pe_vec)

            start_out(g, b)

            @pl.when(g + _NBUF < _NCH)
            def _():
                start_in(g + _NBUF, b)
        return 0

    lax.fori_loop(0, _NCH // _NBUF, outer, 0)
    for b in range(_NBUF):
        wait_out(b)


def kernel(emb, pe_table):
    sc_kernel = pl.kernel(
        _sc_body,
        out_type=jax.ShapeDtypeStruct((_S, _B, _D), jnp.float32),
        mesh=plsc.VectorSubcoreMesh(core_axis_name="c", subcore_axis_name="s"),
        scratch_types=[
            [pltpu.VMEM((_C, _B, _D), jnp.float32) for _ in range(_NBUF)],
            [pltpu.VMEM((_C, _D), jnp.float32) for _ in range(_NBUF)],
            [pltpu.VMEM((_C, _B, _D), jnp.float32) for _ in range(_NBUF)],
            [pltpu.SemaphoreType.DMA for _ in range(_NBUF)],
            [pltpu.SemaphoreType.DMA for _ in range(_NBUF)],
        ],
    )
    return sc_kernel(emb, pe_table)
